# baseline (speedup 1.0000x reference)
import jax
import jax.numpy as jnp
from jax import lax
from jax.experimental import pallas as pl
from jax.experimental.pallas import tpu as pltpu
from jax.experimental.pallas import tpu_sc as plsc

B = 16384
EMB = 32
SHADOW = 32
NC = 2
NS = 16
NW = NC * NS
RPW = B // NW
NGROUPS = RPW // 16
NBROW = 7813
DEPTH = 8
IDPAD = RPW + 32


def _sc_body(uid_hbm, iid_hbm, uT, iT, sT, usT, ub128, ib128, mean_hbm,
             out_hbm,
             uids_v, iids_v, uhi_v, ihi_v, us_all,
             uw, iw, sw, bbuf,
             mean_v, out_v, sem, bsem):
    wid = lax.axis_index("s") * NC + lax.axis_index("c")
    base = wid * RPW

    pltpu.sync_copy(uid_hbm.at[pl.ds(base, RPW)], uids_v.at[pl.ds(0, RPW)])
    pltpu.sync_copy(iid_hbm.at[pl.ds(base, RPW)], iids_v.at[pl.ds(0, RPW)])
    zeros16i = jnp.zeros((16,), jnp.int32)
    uids_v[pl.ds(RPW, 16)] = zeros16i
    uids_v[pl.ds(RPW + 16, 16)] = zeros16i
    iids_v[pl.ds(RPW, 16)] = zeros16i
    iids_v[pl.ds(RPW + 16, 16)] = zeros16i
    mean_v[...] = jnp.zeros((16,), jnp.float32)
    pltpu.sync_copy(mean_hbm, mean_v.at[pl.ds(0, 1)])
    for j in range(4):
        pltpu.sync_copy(usT.at[:, pl.ds(base + j * 128, 128)],
                        us_all.at[pl.ds(j * EMB, EMB), :])

    iota16 = lax.iota(jnp.int32, 16)
    c0 = iota16
    c1 = iota16 + 16
    perms = [jnp.bitwise_xor(iota16, k) for k in (8, 4, 2, 1)]

    def fire(u, i, p):
        offu = pl.multiple_of(lax.shift_right_logical(u, 7) * 128, 128)
        offi = pl.multiple_of(lax.shift_right_logical(i, 7) * 128, 128)
        pltpu.async_copy(uT.at[:, pl.ds(offu, 128)], uw.at[p], sem)
        pltpu.async_copy(iT.at[:, pl.ds(offi, 128)], iw.at[p], sem)
        pltpu.async_copy(sT.at[:, pl.ds(offi, 128)], sw.at[p], sem)

    def wait(p):
        pltpu.make_async_copy(uT.at[:, pl.ds(0, 128)], uw.at[p], sem).wait()
        pltpu.make_async_copy(iT.at[:, pl.ds(0, 128)], iw.at[p], sem).wait()
        pltpu.make_async_copy(sT.at[:, pl.ds(0, 128)], sw.at[p], sem).wait()

    uvec0 = uids_v[pl.ds(0, 16)]
    ivec0 = iids_v[pl.ds(0, 16)]
    for l in range(DEPTH - 1):
        fire(uvec0[l], ivec0[l], l)

    mval = mean_v[...][0]

    def group(g, carry):
        uvec = uids_v[pl.ds(g * 16, 16)]
        ivec = iids_v[pl.ds(g * 16, 16)]
        uvecn = uids_v[pl.ds(g * 16 + 16, 16)]
        ivecn = iids_v[pl.ds(g * 16 + 16, 16)]
        uhi_v[pl.ds(g * 16, 16)] = lax.shift_right_logical(uvec, 7)
        ihi_v[pl.ds(g * 16, 16)] = lax.shift_right_logical(ivec, 7)

        outv = mval + jnp.zeros((16,), jnp.float32)
        for l in range(16):
            p = l % DEPTH
            wait(p)
            la = l + DEPTH - 1
            if la < 16:
                fire(uvec[la], ivec[la], la % DEPTH)
            else:
                fire(uvecn[la - 16], ivecn[la - 16], la % DEPTH)
            u = uvec[l]
            i = ivec[l]
            lane_u = jnp.bitwise_and(u, 127) + zeros16i
            lane_i = jnp.bitwise_and(i, 127) + zeros16i
            b = g * 16 + l
            bhi = lax.shift_right_logical(b, 7) * EMB
            lane_b = jnp.bitwise_and(b, 127) + zeros16i
            u0 = plsc.load_gather(uw.at[p], [c0, lane_u])
            u1 = plsc.load_gather(uw.at[p], [c1, lane_u])
            i0 = plsc.load_gather(iw.at[p], [c0, lane_i])
            i1 = plsc.load_gather(iw.at[p], [c1, lane_i])
            s0 = plsc.load_gather(sw.at[p], [c0, lane_i])
            s1 = plsc.load_gather(sw.at[p], [c1, lane_i])
            us0 = plsc.load_gather(us_all, [bhi + c0, lane_b])
            us1 = plsc.load_gather(us_all, [bhi + c1, lane_b])
            prod = u0 * i0 + u1 * i1 + us0 * s0 + us1 * s1
            for pm in perms:
                prod = prod + prod.at[pm].get(mode="promise_in_bounds",
                                              unique_indices=True)
            outv = jnp.where(iota16 == l, prod, outv)
        out_v[pl.ds(g * 16, 16)] = outv
        return carry

    lax.fori_loop(0, NGROUPS, group, 0)
    for l in range(DEPTH - 1):
        wait((RPW + l) % DEPTH)

    for hi_v, ids_v, tab in ((uhi_v, uids_v, ub128), (ihi_v, iids_v, ib128)):
        for j in range(8):
            pltpu.async_copy(tab.at[hi_v.at[pl.ds(j * 64, 64)]],
                             bbuf, bsem).wait()

            def badd(g2, carry):
                m = g2 * 16 + iota16
                s = pl.ds(j * 64 + g2 * 16, 16)
                idv = ids_v[s]
                bv = plsc.load_gather(bbuf, [m, jnp.bitwise_and(idv, 127)])
                out_v[s] = out_v[s] + bv
                return carry

            lax.fori_loop(0, 4, badd, 0)

    pltpu.sync_copy(out_v, out_hbm.at[pl.ds(base, RPW)])


@jax.jit
def _run(uid, iid, uT, iT, sT, usT, ub128, ib128, mean):
    mesh = plsc.VectorSubcoreMesh(
        core_axis_name="c", subcore_axis_name="s",
        num_cores=NC, num_subcores=NS)
    f = pl.kernel(
        _sc_body,
        out_type=jax.ShapeDtypeStruct((B,), jnp.float32),
        mesh=mesh,
        scratch_types=[
            pltpu.VMEM((IDPAD,), jnp.int32),
            pltpu.VMEM((IDPAD,), jnp.int32),
            pltpu.VMEM((RPW,), jnp.int32),
            pltpu.VMEM((RPW,), jnp.int32),
            pltpu.VMEM((4 * EMB, 128), jnp.float32),
            pltpu.VMEM((DEPTH, EMB, 128), jnp.float32),
            pltpu.VMEM((DEPTH, EMB, 128), jnp.float32),
            pltpu.VMEM((DEPTH, EMB, 128), jnp.float32),
            pltpu.VMEM((64, 128), jnp.float32),
            pltpu.VMEM((16,), jnp.float32),
            pltpu.VMEM((RPW,), jnp.float32),
            pltpu.SemaphoreType.DMA,
            pltpu.SemaphoreType.DMA,
        ],
        compiler_params=pltpu.CompilerParams(
            needs_layout_passes=False, use_tc_tiling_on_sc=True),
    )
    return f(uid, iid, uT, iT, sT, usT, ub128, ib128, mean)


def kernel(u_id, i_id, UserShadow, user_emb, user_bias, item_emb, item_bias,
           shadow_i_emb, mean):
    uid = u_id.astype(jnp.int32)
    iid = i_id.astype(jnp.int32)
    ub128 = jnp.pad(user_bias.reshape(-1), (0, NBROW * 128 - 1000000)
                    ).reshape(NBROW, 128)
    ib128 = jnp.pad(item_bias.reshape(-1), (0, NBROW * 128 - 1000000)
                    ).reshape(NBROW, 128)
    return _run(uid, iid, user_emb.T, item_emb.T, shadow_i_emb.T,
                UserShadow.T, ub128, ib128, mean)


# speedup vs baseline: 1.0485x; 1.0485x over previous
import jax
import jax.numpy as jnp
from jax import lax
from jax.experimental import pallas as pl
from jax.experimental.pallas import tpu as pltpu
from jax.experimental.pallas import tpu_sc as plsc

B = 16384
EMB = 32
SHADOW = 32
NC = 2
NS = 16
NW = NC * NS
RPW = B // NW
NGROUPS = RPW // 16
NBROW = 7813
DEPTH = 4
IDPAD = RPW + 32


def _sc_body(uid_hbm, iid_hbm, uT, iT, sT, usT, ub128, ib128, mean_hbm,
             out_hbm,
             uids_v, iids_v, uhi_v, ihi_v, us_all,
             uw, iw, sw, bbuf_u, bbuf_i,
             mean_v, out_v, sem, bsem):
    wid = lax.axis_index("s") * NC + lax.axis_index("c")
    base = wid * RPW

    pltpu.sync_copy(uid_hbm.at[pl.ds(base, RPW)], uids_v.at[pl.ds(0, RPW)])
    pltpu.sync_copy(iid_hbm.at[pl.ds(base, RPW)], iids_v.at[pl.ds(0, RPW)])
    zeros16i = jnp.zeros((16,), jnp.int32)
    uids_v[pl.ds(RPW, 16)] = zeros16i
    uids_v[pl.ds(RPW + 16, 16)] = zeros16i
    iids_v[pl.ds(RPW, 16)] = zeros16i
    iids_v[pl.ds(RPW + 16, 16)] = zeros16i
    mean_v[...] = jnp.zeros((16,), jnp.float32)
    pltpu.sync_copy(mean_hbm, mean_v.at[pl.ds(0, 1)])
    for j in range(4):
        pltpu.sync_copy(usT.at[:, pl.ds(base + j * 128, 128)],
                        us_all.at[pl.ds(j * EMB, EMB), :])

    iota16 = lax.iota(jnp.int32, 16)
    c0 = iota16
    c1 = iota16 + 16
    perms = [jnp.bitwise_xor(iota16, k) for k in (8, 4, 2, 1)]

    def fire(u, i, p):
        offu = pl.multiple_of(lax.shift_right_logical(u, 7) * 128, 128)
        offi = pl.multiple_of(lax.shift_right_logical(i, 7) * 128, 128)
        pltpu.async_copy(uT.at[:, pl.ds(offu, 128)], uw.at[p], sem)
        pltpu.async_copy(iT.at[:, pl.ds(offi, 128)], iw.at[p], sem)
        pltpu.async_copy(sT.at[:, pl.ds(offi, 128)], sw.at[p], sem)

    def wait(p):
        pltpu.make_async_copy(uT.at[:, pl.ds(0, 128)], uw.at[p], sem).wait()
        pltpu.make_async_copy(iT.at[:, pl.ds(0, 128)], iw.at[p], sem).wait()
        pltpu.make_async_copy(sT.at[:, pl.ds(0, 128)], sw.at[p], sem).wait()

    uvec0 = uids_v[pl.ds(0, 16)]
    ivec0 = iids_v[pl.ds(0, 16)]
    for l in range(DEPTH - 1):
        fire(uvec0[l], ivec0[l], l)

    mval = mean_v[...][0]

    def group(g, carry):
        uvec = uids_v[pl.ds(g * 16, 16)]
        ivec = iids_v[pl.ds(g * 16, 16)]
        uvecn = uids_v[pl.ds(g * 16 + 16, 16)]
        ivecn = iids_v[pl.ds(g * 16 + 16, 16)]
        uhi_v[pl.ds(g * 16, 16)] = lax.shift_right_logical(uvec, 7)
        ihi_v[pl.ds(g * 16, 16)] = lax.shift_right_logical(ivec, 7)

        outv = mval + jnp.zeros((16,), jnp.float32)
        for l in range(16):
            p = l % DEPTH
            wait(p)
            la = l + DEPTH - 1
            if la < 16:
                fire(uvec[la], ivec[la], la % DEPTH)
            else:
                fire(uvecn[la - 16], ivecn[la - 16], la % DEPTH)
            u = uvec[l]
            i = ivec[l]
            lane_u = jnp.bitwise_and(u, 127) + zeros16i
            lane_i = jnp.bitwise_and(i, 127) + zeros16i
            b = g * 16 + l
            bhi = lax.shift_right_logical(b, 7) * EMB
            lane_b = jnp.bitwise_and(b, 127) + zeros16i
            u0 = plsc.load_gather(uw.at[p], [c0, lane_u])
            u1 = plsc.load_gather(uw.at[p], [c1, lane_u])
            i0 = plsc.load_gather(iw.at[p], [c0, lane_i])
            i1 = plsc.load_gather(iw.at[p], [c1, lane_i])
            s0 = plsc.load_gather(sw.at[p], [c0, lane_i])
            s1 = plsc.load_gather(sw.at[p], [c1, lane_i])
            us0 = plsc.load_gather(us_all, [bhi + c0, lane_b])
            us1 = plsc.load_gather(us_all, [bhi + c1, lane_b])
            prod = u0 * i0 + u1 * i1 + us0 * s0 + us1 * s1
            for pm in perms:
                prod = prod + prod.at[pm].get(mode="promise_in_bounds",
                                              unique_indices=True)
            outv = jnp.where(iota16 == l, prod, outv)
        out_v[pl.ds(g * 16, 16)] = outv
        return carry

    lax.fori_loop(0, NGROUPS, group, 0)
    for l in range(DEPTH - 1):
        wait((RPW + l) % DEPTH)

    for j in range(4):
        cu = pltpu.async_copy(ub128.at[uhi_v.at[pl.ds(j * 128, 128)]],
                              bbuf_u, bsem)
        ci = pltpu.async_copy(ib128.at[ihi_v.at[pl.ds(j * 128, 128)]],
                              bbuf_i, bsem)
        cu.wait()
        ci.wait()

        def badd(g2, carry):
            m = g2 * 16 + iota16
            s = pl.ds(j * 128 + g2 * 16, 16)
            uvec = uids_v[s]
            ivec = iids_v[s]
            bu = plsc.load_gather(bbuf_u, [m, jnp.bitwise_and(uvec, 127)])
            bi = plsc.load_gather(bbuf_i, [m, jnp.bitwise_and(ivec, 127)])
            out_v[s] = out_v[s] + bu + bi
            return carry

        lax.fori_loop(0, 8, badd, 0)

    pltpu.sync_copy(out_v, out_hbm.at[pl.ds(base, RPW)])


@jax.jit
def _run(uid, iid, uT, iT, sT, usT, ub128, ib128, mean):
    mesh = plsc.VectorSubcoreMesh(
        core_axis_name="c", subcore_axis_name="s",
        num_cores=NC, num_subcores=NS)
    f = pl.kernel(
        _sc_body,
        out_type=jax.ShapeDtypeStruct((B,), jnp.float32),
        mesh=mesh,
        scratch_types=[
            pltpu.VMEM((IDPAD,), jnp.int32),
            pltpu.VMEM((IDPAD,), jnp.int32),
            pltpu.VMEM((RPW,), jnp.int32),
            pltpu.VMEM((RPW,), jnp.int32),
            pltpu.VMEM((4 * EMB, 128), jnp.float32),
            pltpu.VMEM((DEPTH, EMB, 128), jnp.float32),
            pltpu.VMEM((DEPTH, EMB, 128), jnp.float32),
            pltpu.VMEM((DEPTH, EMB, 128), jnp.float32),
            pltpu.VMEM((128, 128), jnp.float32),
            pltpu.VMEM((128, 128), jnp.float32),
            pltpu.VMEM((16,), jnp.float32),
            pltpu.VMEM((RPW,), jnp.float32),
            pltpu.SemaphoreType.DMA,
            pltpu.SemaphoreType.DMA,
        ],
        compiler_params=pltpu.CompilerParams(
            needs_layout_passes=False, use_tc_tiling_on_sc=True),
    )
    return f(uid, iid, uT, iT, sT, usT, ub128, ib128, mean)


def kernel(u_id, i_id, UserShadow, user_emb, user_bias, item_emb, item_bias,
           shadow_i_emb, mean):
    uid = u_id.astype(jnp.int32)
    iid = i_id.astype(jnp.int32)
    ub128 = jnp.pad(user_bias.reshape(-1), (0, NBROW * 128 - 1000000)
                    ).reshape(NBROW, 128)
    ib128 = jnp.pad(item_bias.reshape(-1), (0, NBROW * 128 - 1000000)
                    ).reshape(NBROW, 128)
    return _run(uid, iid, user_emb.T, item_emb.T, shadow_i_emb.T,
                UserShadow.T, ub128, ib128, mean)
